# baseline (speedup 1.0000x reference)
import functools

import jax
import jax.numpy as jnp
from jax import lax
from jax.experimental import pallas as pl
from jax.experimental.pallas import tpu as pltpu
from jax.experimental.pallas import tpu_sc as plsc

N_NODES = 10000
N_EDGES = 320000
NFEAT = 128
NHID = 128
NCLASS = 40

NC = 2
NS = 16
NW = NC * NS
EDGES_PER_TILE = N_EDGES // NW
DCHUNK = 80
DNCHUNK = EDGES_PER_TILE // DCHUNK
ROWS_PER_TILE = N_NODES // NS
ZR = 25

CHUNK = 128
NCHUNK = 80
EPT = NCHUNK * CHUNK
N_ACC = 10240
ACC_PER_TILE = N_ACC // NS
AZR = 32

_mesh = functools.partial(
    plsc.VectorSubcoreMesh, core_axis_name="c", subcore_axis_name="s")


def _zero_fill(zb, d):
    z16 = jnp.zeros((16,), jnp.float32)
    for r in range(ZR):
        for q in range(d // 16):
            zb[r, pl.ds(16 * q, 16)] = z16


def _zero_acc(zb, acc, s):
    for j in range(ROWS_PER_TILE // ZR):
        pltpu.sync_copy(zb, acc.at[pl.ds(s * ROWS_PER_TILE + j * ZR, ZR)])


def _copy_out(acc, out, c, s):
    pltpu.sync_copy(acc.at[pl.ds(s * ROWS_PER_TILE, ROWS_PER_TILE)],
                    out.at[c, s])


def _deg_body(dst_hbm, out_hbm, dst_v, ones_v, zb, acc):
    c = lax.axis_index("c")
    s = lax.axis_index("s")
    wid = s * NC + c
    o16 = jnp.ones((16,), jnp.float32)
    for r in range(DCHUNK):
        for q in range(NHID // 16):
            ones_v[r, pl.ds(16 * q, 16)] = o16
    _zero_fill(zb, NHID)
    _zero_acc(zb, acc, s)
    pltpu.sync_copy(dst_hbm.at[wid], dst_v)
    plsc.subcore_barrier()

    def body(i, carry):
        pltpu.sync_copy(ones_v, acc.at[dst_v.at[i]], add=True)
        return carry

    lax.fori_loop(0, DNCHUNK, body, 0)
    plsc.subcore_barrier()
    _copy_out(acc, out_hbm, c, s)


def _deg(dst3):
    kern = pl.kernel(
        _deg_body,
        out_type=jax.ShapeDtypeStruct((NC, NS, ROWS_PER_TILE, NHID),
                                      jnp.float32),
        mesh=_mesh(),
        scratch_types=[
            pltpu.VMEM((DNCHUNK, DCHUNK), jnp.int32),
            pltpu.VMEM((DCHUNK, NHID), jnp.float32),
            pltpu.VMEM((ZR, NHID), jnp.float32),
            pltpu.VMEM_SHARED((N_NODES, NHID), jnp.float32),
        ],
    )
    return kern(dst3).reshape(NC, N_NODES, NHID)


def _spmm_body(d, feat_hbm, src_hbm, dst_hbm, out_hbm,
               src_a, src_b, dst_v, rows_a, rows_b, acc,
               sem_sa, sem_sb, sem_ra, sem_rb):
    c = lax.axis_index("c")
    s = lax.axis_index("s")
    wid = s * NC + c
    pltpu.sync_copy(dst_hbm.at[wid], dst_v)
    z16 = jnp.zeros((16,), jnp.float32)
    for r in range(AZR):
        for q in range(d // 16):
            rows_a[r, pl.ds(16 * q, 16)] = z16
    for j in range(ACC_PER_TILE // AZR):
        pltpu.sync_copy(rows_a.at[pl.ds(0, AZR)],
                        acc.at[pl.ds(s * ACC_PER_TILE + j * AZR, AZR)])
    pltpu.sync_copy(src_hbm.at[wid, 0], src_a)
    pltpu.make_async_copy(feat_hbm.at[src_a.at[0]], rows_a, sem_ra).start()
    pltpu.make_async_copy(src_hbm.at[wid, 1], src_b, sem_sb).start()
    plsc.subcore_barrier()

    def step(i, sbuf_x, sbuf_y, rbuf_x, rbuf_y, ssem_x, ssem_y,
             rsem_x, rsem_y):
        pltpu.make_async_copy(src_hbm.at[wid, i + 1], sbuf_y, ssem_y).wait()
        pltpu.make_async_copy(
            feat_hbm.at[sbuf_y.at[0]], rbuf_y, rsem_y).start()
        pltpu.make_async_copy(
            feat_hbm.at[sbuf_x.at[0]], rbuf_x, rsem_x).wait()
        pltpu.make_async_copy(src_hbm.at[wid, i + 2], sbuf_x, ssem_x).start()
        pltpu.sync_copy(rbuf_x, acc.at[dst_v.at[i]], add=True)

    def body(k, carry):
        i0 = k * 2
        step(i0, src_a, src_b, rows_a, rows_b, sem_sa, sem_sb,
             sem_ra, sem_rb)
        step(i0 + 1, src_b, src_a, rows_b, rows_a, sem_sb, sem_sa,
             sem_rb, sem_ra)
        return carry

    lax.fori_loop(0, NCHUNK // 2 - 1, body, 0)
    pltpu.make_async_copy(
        src_hbm.at[wid, NCHUNK - 1], src_b, sem_sb).wait()
    pltpu.make_async_copy(
        feat_hbm.at[src_b.at[0]], rows_b, sem_rb).start()
    pltpu.make_async_copy(
        feat_hbm.at[src_a.at[0]], rows_a, sem_ra).wait()
    pltpu.sync_copy(rows_a, acc.at[dst_v.at[NCHUNK - 2]], add=True)
    pltpu.make_async_copy(
        feat_hbm.at[src_b.at[0]], rows_b, sem_rb).wait()
    pltpu.sync_copy(rows_b, acc.at[dst_v.at[NCHUNK - 1]], add=True)
    plsc.subcore_barrier()
    pltpu.sync_copy(acc.at[pl.ds(s * ACC_PER_TILE, ACC_PER_TILE)],
                    out_hbm.at[c, s])


def _spmm(feat, srcp, dstp, d):
    kern = pl.kernel(
        functools.partial(_spmm_body, d),
        out_type=jax.ShapeDtypeStruct((NC, NS, ACC_PER_TILE, d),
                                      jnp.float32),
        mesh=_mesh(),
        scratch_types=[
            pltpu.VMEM((1, CHUNK), jnp.int32),
            pltpu.VMEM((1, CHUNK), jnp.int32),
            pltpu.VMEM((NCHUNK, CHUNK), jnp.int32),
            pltpu.VMEM((CHUNK, d), jnp.float32),
            pltpu.VMEM((CHUNK, d), jnp.float32),
            pltpu.VMEM_SHARED((N_ACC, d), jnp.float32),
            pltpu.SemaphoreType.DMA,
            pltpu.SemaphoreType.DMA,
            pltpu.SemaphoreType.DMA,
            pltpu.SemaphoreType.DMA,
        ],
    )
    out = kern(feat, srcp, dstp).reshape(NC, NS * ACC_PER_TILE, d)
    return out[:, :N_NODES]


ROWS_TC = 2000


def _scale_in_body(x_ref, w1_ref, degc_ref, xws_ref, dinv_ref):
    deg = jnp.maximum(degc_ref[0] + degc_ref[1], 1.0)
    dinv = lax.rsqrt(deg)
    dinv_ref[...] = dinv
    xw = jnp.dot(x_ref[...], w1_ref[...],
                 preferred_element_type=jnp.float32)
    xws_ref[...] = xw * dinv[:, 0:1]


def _mid_body(p_ref, dinv_ref, out_ref):
    dv = dinv_ref[:, 0:1]
    h = jnp.maximum((p_ref[0] + p_ref[1]) * dv, 0.0)
    out_ref[...] = h * dv


def _final_body(q_ref, dinv_ref, w2_ref, b2_ref, out_ref):
    z = (q_ref[0] + q_ref[1]) * dinv_ref[:, 0:1]
    logits = jnp.dot(z, w2_ref[...],
                     preferred_element_type=jnp.float32) + b2_ref[0:1, :]
    mx = jnp.max(logits, axis=1, keepdims=True)
    lse = jnp.log(jnp.sum(jnp.exp(logits - mx), axis=1, keepdims=True)) + mx
    out_ref[...] = logits - lse


def kernel(x, edge_index, W1, W2, b2):
    ei = edge_index.astype(jnp.int32)
    dst3 = ei[1].reshape(NW, DNCHUNK, DCHUNK)
    pad = EPT - EDGES_PER_TILE
    src2 = ei[0].reshape(NW, EDGES_PER_TILE)
    dst2 = ei[1].reshape(NW, EDGES_PER_TILE)
    srcp = jnp.pad(src2, ((0, 0), (0, pad))).reshape(NW, NCHUNK, 1, CHUNK)
    dstp = jnp.pad(dst2, ((0, 0), (0, pad)),
                   constant_values=N_NODES).reshape(NW, NCHUNK, CHUNK)
    b2r = b2.reshape(1, NCLASS)

    degc = _deg(dst3)[:, :, :16]

    grid = (N_NODES // ROWS_TC,)
    xws, dinv16 = pl.pallas_call(
        _scale_in_body,
        grid=grid,
        in_specs=[
            pl.BlockSpec((ROWS_TC, NFEAT), lambda i: (i, 0)),
            pl.BlockSpec((NFEAT, NHID), lambda i: (0, 0)),
            pl.BlockSpec((NC, ROWS_TC, 16), lambda i: (0, i, 0)),
        ],
        out_specs=[
            pl.BlockSpec((ROWS_TC, NHID), lambda i: (i, 0)),
            pl.BlockSpec((ROWS_TC, 16), lambda i: (i, 0)),
        ],
        out_shape=[
            jax.ShapeDtypeStruct((N_NODES, NHID), jnp.float32),
            jax.ShapeDtypeStruct((N_NODES, 16), jnp.float32),
        ],
    )(x, W1, degc)

    p = _spmm(xws, srcp, dstp, NHID)

    hs = pl.pallas_call(
        _mid_body,
        grid=grid,
        in_specs=[
            pl.BlockSpec((NC, ROWS_TC, NHID), lambda i: (0, i, 0)),
            pl.BlockSpec((ROWS_TC, 16), lambda i: (i, 0)),
        ],
        out_specs=pl.BlockSpec((ROWS_TC, NHID), lambda i: (i, 0)),
        out_shape=jax.ShapeDtypeStruct((N_NODES, NHID), jnp.float32),
    )(p, dinv16)

    q = _spmm(hs, srcp, dstp, NHID)

    out = pl.pallas_call(
        _final_body,
        grid=grid,
        in_specs=[
            pl.BlockSpec((NC, ROWS_TC, NHID), lambda i: (0, i, 0)),
            pl.BlockSpec((ROWS_TC, 16), lambda i: (i, 0)),
            pl.BlockSpec((NHID, NCLASS), lambda i: (0, 0)),
            pl.BlockSpec((1, NCLASS), lambda i: (0, 0)),
        ],
        out_specs=pl.BlockSpec((ROWS_TC, NCLASS), lambda i: (i, 0)),
        out_shape=jax.ShapeDtypeStruct((N_NODES, NCLASS), jnp.float32),
    )(q, dinv16, W2, b2r)
    return out


# speedup vs baseline: 2.2646x; 2.2646x over previous
import functools

import jax
import jax.numpy as jnp
from jax import lax
from jax.experimental import pallas as pl
from jax.experimental.pallas import tpu as pltpu
from jax.experimental.pallas import tpu_sc as plsc

N_NODES = 10000
N_EDGES = 320000
NFEAT = 128
NHID = 128
NCLASS = 40

NC = 2
NS = 16
NW = NC * NS
EDGES_PER_TILE = N_EDGES // NW
DCHUNK = 80
DNCHUNK = EDGES_PER_TILE // DCHUNK
ROWS_PER_TILE = N_NODES // NS
ZR = 25

CHUNK = 80
NCHUNK = EDGES_PER_TILE // CHUNK
PAGES = 5
PCHUNK = NCHUNK // PAGES

_mesh = functools.partial(
    plsc.VectorSubcoreMesh, core_axis_name="c", subcore_axis_name="s")


def _zero_fill(zb, d):
    z16 = jnp.zeros((16,), jnp.float32)
    for r in range(ZR):
        for q in range(d // 16):
            zb[r, pl.ds(16 * q, 16)] = z16


def _zero_acc(zb, acc, s):
    for j in range(ROWS_PER_TILE // ZR):
        pltpu.sync_copy(zb, acc.at[pl.ds(s * ROWS_PER_TILE + j * ZR, ZR)])


def _copy_out(acc, out, c, s):
    pltpu.sync_copy(acc.at[pl.ds(s * ROWS_PER_TILE, ROWS_PER_TILE)],
                    out.at[c, s])


def _deg_body(dst_hbm, out_hbm, dst_v, ones_v, zb, acc):
    c = lax.axis_index("c")
    s = lax.axis_index("s")
    wid = s * NC + c
    o16 = jnp.ones((16,), jnp.float32)
    for r in range(DCHUNK):
        for q in range(NHID // 16):
            ones_v[r, pl.ds(16 * q, 16)] = o16
    _zero_fill(zb, NHID)
    _zero_acc(zb, acc, s)
    pltpu.sync_copy(dst_hbm.at[wid], dst_v)
    plsc.subcore_barrier()

    def body(i, carry):
        pltpu.sync_copy(ones_v, acc.at[dst_v.at[i]], add=True)
        return carry

    lax.fori_loop(0, DNCHUNK, body, 0)
    plsc.subcore_barrier()
    _copy_out(acc, out_hbm, c, s)


def _deg(dst3):
    kern = pl.kernel(
        _deg_body,
        out_type=jax.ShapeDtypeStruct((NC, NS, ROWS_PER_TILE, NHID),
                                      jnp.float32),
        mesh=_mesh(),
        scratch_types=[
            pltpu.VMEM((DNCHUNK, DCHUNK), jnp.int32),
            pltpu.VMEM((DCHUNK, NHID), jnp.float32),
            pltpu.VMEM((ZR, NHID), jnp.float32),
            pltpu.VMEM_SHARED((N_NODES, NHID), jnp.float32),
        ],
    )
    return kern(dst3).reshape(NC, N_NODES, NHID)


def _spmm_body(d, feat_hbm, src_hbm, dst_hbm, out_hbm,
               src_pg, dst_pg, rows_a, rows_b, acc, sem_ra, sem_rb):
    c = lax.axis_index("c")
    s = lax.axis_index("s")
    wid = s * NC + c
    z16 = jnp.zeros((16,), jnp.float32)
    for r in range(ZR):
        for q in range(d // 16):
            rows_a[r, pl.ds(16 * q, 16)] = z16
    for j in range(ROWS_PER_TILE // ZR):
        pltpu.sync_copy(rows_a.at[pl.ds(0, ZR)],
                        acc.at[pl.ds(s * ROWS_PER_TILE + j * ZR, ZR)])
    plsc.subcore_barrier()

    def step(i, rbuf_x, rbuf_y, rsem_x, rsem_y):
        pltpu.make_async_copy(
            feat_hbm.at[src_pg.at[i + 1]], rbuf_y, rsem_y).start()
        pltpu.make_async_copy(
            feat_hbm.at[src_pg.at[i]], rbuf_x, rsem_x).wait()
        pltpu.sync_copy(rbuf_x, acc.at[dst_pg.at[i]], add=True)

    def body(k, carry):
        i0 = k * 2
        step(i0, rows_a, rows_b, sem_ra, sem_rb)
        step(i0 + 1, rows_b, rows_a, sem_rb, sem_ra)
        return carry

    for p in range(PAGES):
        pltpu.sync_copy(src_hbm.at[wid, p], src_pg)
        pltpu.sync_copy(dst_hbm.at[wid, p], dst_pg)
        pltpu.make_async_copy(
            feat_hbm.at[src_pg.at[0]], rows_a, sem_ra).start()
        lax.fori_loop(0, (PCHUNK - 1) // 2, body, 0)
        pltpu.make_async_copy(
            feat_hbm.at[src_pg.at[PCHUNK - 1]], rows_a, sem_ra).wait()
        pltpu.sync_copy(rows_a, acc.at[dst_pg.at[PCHUNK - 1]], add=True)

    plsc.subcore_barrier()
    _copy_out(acc, out_hbm, c, s)


def _spmm(feat, srcp, dstp, d):
    kern = pl.kernel(
        functools.partial(_spmm_body, d),
        out_type=jax.ShapeDtypeStruct((NC, NS, ROWS_PER_TILE, d),
                                      jnp.float32),
        mesh=_mesh(),
        scratch_types=[
            pltpu.VMEM((PCHUNK, CHUNK), jnp.int32),
            pltpu.VMEM((PCHUNK, CHUNK), jnp.int32),
            pltpu.VMEM((CHUNK, d), jnp.float32),
            pltpu.VMEM((CHUNK, d), jnp.float32),
            pltpu.VMEM_SHARED((N_NODES, d), jnp.float32),
            pltpu.SemaphoreType.DMA,
            pltpu.SemaphoreType.DMA,
        ],
    )
    return kern(feat, srcp, dstp).reshape(NC, N_NODES, d)


ROWS_TC = 2000


def _scale_in_body(x_ref, w1_ref, degc_ref, xws_ref, dinv_ref):
    deg = jnp.maximum(degc_ref[0] + degc_ref[1], 1.0)
    dinv = lax.rsqrt(deg)
    dinv_ref[...] = dinv
    xw = jnp.dot(x_ref[...], w1_ref[...],
                 preferred_element_type=jnp.float32)
    xws_ref[...] = xw * dinv[:, 0:1]


def _mid_body(p_ref, dinv_ref, out_ref):
    dv = dinv_ref[:, 0:1]
    h = jnp.maximum((p_ref[0] + p_ref[1]) * dv, 0.0)
    out_ref[...] = h * dv


def _final_body(q_ref, dinv_ref, w2_ref, b2_ref, out_ref):
    z = (q_ref[0] + q_ref[1]) * dinv_ref[:, 0:1]
    logits = jnp.dot(z, w2_ref[...],
                     preferred_element_type=jnp.float32) + b2_ref[0:1, :]
    mx = jnp.max(logits, axis=1, keepdims=True)
    lse = jnp.log(jnp.sum(jnp.exp(logits - mx), axis=1, keepdims=True)) + mx
    out_ref[...] = logits - lse


def kernel(x, edge_index, W1, W2, b2):
    ei = edge_index.astype(jnp.int32)
    dst3 = ei[1].reshape(NW, DNCHUNK, DCHUNK)
    srcp = ei[0].reshape(NW, PAGES, PCHUNK, CHUNK)
    dstp = ei[1].reshape(NW, PAGES, PCHUNK, CHUNK)
    b2r = b2.reshape(1, NCLASS)

    degc = _deg(dst3)[:, :, :16]

    grid = (N_NODES // ROWS_TC,)
    xws, dinv16 = pl.pallas_call(
        _scale_in_body,
        grid=grid,
        in_specs=[
            pl.BlockSpec((ROWS_TC, NFEAT), lambda i: (i, 0)),
            pl.BlockSpec((NFEAT, NHID), lambda i: (0, 0)),
            pl.BlockSpec((NC, ROWS_TC, 16), lambda i: (0, i, 0)),
        ],
        out_specs=[
            pl.BlockSpec((ROWS_TC, NHID), lambda i: (i, 0)),
            pl.BlockSpec((ROWS_TC, 16), lambda i: (i, 0)),
        ],
        out_shape=[
            jax.ShapeDtypeStruct((N_NODES, NHID), jnp.float32),
            jax.ShapeDtypeStruct((N_NODES, 16), jnp.float32),
        ],
    )(x, W1, degc)

    p = _spmm(xws, srcp, dstp, NHID)

    hs = pl.pallas_call(
        _mid_body,
        grid=grid,
        in_specs=[
            pl.BlockSpec((NC, ROWS_TC, NHID), lambda i: (0, i, 0)),
            pl.BlockSpec((ROWS_TC, 16), lambda i: (i, 0)),
        ],
        out_specs=pl.BlockSpec((ROWS_TC, NHID), lambda i: (i, 0)),
        out_shape=jax.ShapeDtypeStruct((N_NODES, NHID), jnp.float32),
    )(p, dinv16)

    q = _spmm(hs, srcp, dstp, NHID)

    out = pl.pallas_call(
        _final_body,
        grid=grid,
        in_specs=[
            pl.BlockSpec((NC, ROWS_TC, NHID), lambda i: (0, i, 0)),
            pl.BlockSpec((ROWS_TC, 16), lambda i: (i, 0)),
            pl.BlockSpec((NHID, NCLASS), lambda i: (0, 0)),
            pl.BlockSpec((1, NCLASS), lambda i: (0, 0)),
        ],
        out_specs=pl.BlockSpec((ROWS_TC, NCLASS), lambda i: (i, 0)),
        out_shape=jax.ShapeDtypeStruct((N_NODES, NCLASS), jnp.float32),
    )(q, dinv16, W2, b2r)
    return out


# speedup vs baseline: 2.2730x; 1.0037x over previous
import functools

import jax
import jax.numpy as jnp
from jax import lax
from jax.experimental import pallas as pl
from jax.experimental.pallas import tpu as pltpu
from jax.experimental.pallas import tpu_sc as plsc

N_NODES = 10000
N_EDGES = 320000
NFEAT = 128
NHID = 128
NCLASS = 40

NC = 2
NS = 16
NW = NC * NS
EDGES_PER_TILE = N_EDGES // NW
DCHUNK = 80
DNCHUNK = EDGES_PER_TILE // DCHUNK
ROWS_PER_TILE = N_NODES // NS
ZR = 25

CHUNK = 80
NCHUNK = EDGES_PER_TILE // CHUNK
PAGES = 5
PCHUNK = NCHUNK // PAGES

_mesh = functools.partial(
    plsc.VectorSubcoreMesh, core_axis_name="c", subcore_axis_name="s")


def _zero_fill(zb, d):
    z16 = jnp.zeros((16,), jnp.float32)
    for r in range(ZR):
        for q in range(d // 16):
            zb[r, pl.ds(16 * q, 16)] = z16


def _zero_acc(zb, acc, s):
    for j in range(ROWS_PER_TILE // ZR):
        pltpu.sync_copy(zb, acc.at[pl.ds(s * ROWS_PER_TILE + j * ZR, ZR)])


def _copy_out(acc, out, c, s):
    pltpu.sync_copy(acc.at[pl.ds(s * ROWS_PER_TILE, ROWS_PER_TILE)],
                    out.at[c, s])


def _deg_body(dst_hbm, out_hbm, dst_v, ones_v, zb, acc, sem):
    c = lax.axis_index("c")
    s = lax.axis_index("s")
    wid = s * NC + c
    o16 = jnp.ones((16,), jnp.float32)
    for r in range(DCHUNK):
        for q in range(NHID // 16):
            ones_v[r, pl.ds(16 * q, 16)] = o16
    _zero_fill(zb, NHID)
    _zero_acc(zb, acc, s)
    pltpu.sync_copy(dst_hbm.at[wid], dst_v)
    plsc.subcore_barrier()

    def body(i, carry):
        pltpu.async_copy(ones_v, acc.at[dst_v.at[i]], sem, add=True)
        return carry

    lax.fori_loop(0, DNCHUNK, body, 0)

    def drain(i, carry):
        pltpu.make_async_copy(ones_v, acc.at[dst_v.at[i]], sem).wait()
        return carry

    lax.fori_loop(0, DNCHUNK, drain, 0)
    plsc.subcore_barrier()
    _copy_out(acc, out_hbm, c, s)


def _deg(dst3):
    kern = pl.kernel(
        _deg_body,
        out_type=jax.ShapeDtypeStruct((NC, NS, ROWS_PER_TILE, NHID),
                                      jnp.float32),
        mesh=_mesh(),
        scratch_types=[
            pltpu.VMEM((DNCHUNK, DCHUNK), jnp.int32),
            pltpu.VMEM((DCHUNK, NHID), jnp.float32),
            pltpu.VMEM((ZR, NHID), jnp.float32),
            pltpu.VMEM_SHARED((N_NODES, NHID), jnp.float32),
            pltpu.SemaphoreType.DMA,
        ],
    )
    return kern(dst3).reshape(NC, N_NODES, NHID)


def _spmm_body(d, feat_hbm, src_hbm, dst_hbm, out_hbm,
               src_pg, dst_pg, rows_a, rows_b, acc, sem_ra, sem_rb):
    c = lax.axis_index("c")
    s = lax.axis_index("s")
    wid = s * NC + c
    z16 = jnp.zeros((16,), jnp.float32)
    for r in range(ZR):
        for q in range(d // 16):
            rows_a[r, pl.ds(16 * q, 16)] = z16
    for j in range(ROWS_PER_TILE // ZR):
        pltpu.sync_copy(rows_a.at[pl.ds(0, ZR)],
                        acc.at[pl.ds(s * ROWS_PER_TILE + j * ZR, ZR)])
    plsc.subcore_barrier()

    def step(i, rbuf_x, rbuf_y, rsem_x, rsem_y):
        pltpu.make_async_copy(
            feat_hbm.at[src_pg.at[i + 1]], rbuf_y, rsem_y).start()
        pltpu.make_async_copy(
            feat_hbm.at[src_pg.at[i]], rbuf_x, rsem_x).wait()
        pltpu.sync_copy(rbuf_x, acc.at[dst_pg.at[i]], add=True)

    def body(k, carry):
        i0 = k * 2
        step(i0, rows_a, rows_b, sem_ra, sem_rb)
        step(i0 + 1, rows_b, rows_a, sem_rb, sem_ra)
        return carry

    for p in range(PAGES):
        pltpu.sync_copy(src_hbm.at[wid, p], src_pg)
        pltpu.sync_copy(dst_hbm.at[wid, p], dst_pg)
        pltpu.make_async_copy(
            feat_hbm.at[src_pg.at[0]], rows_a, sem_ra).start()
        lax.fori_loop(0, (PCHUNK - 1) // 2, body, 0)
        pltpu.make_async_copy(
            feat_hbm.at[src_pg.at[PCHUNK - 1]], rows_a, sem_ra).wait()
        pltpu.sync_copy(rows_a, acc.at[dst_pg.at[PCHUNK - 1]], add=True)

    plsc.subcore_barrier()
    _copy_out(acc, out_hbm, c, s)


def _spmm(feat, srcp, dstp, d):
    kern = pl.kernel(
        functools.partial(_spmm_body, d),
        out_type=jax.ShapeDtypeStruct((NC, NS, ROWS_PER_TILE, d),
                                      jnp.float32),
        mesh=_mesh(),
        scratch_types=[
            pltpu.VMEM((PCHUNK, CHUNK), jnp.int32),
            pltpu.VMEM((PCHUNK, CHUNK), jnp.int32),
            pltpu.VMEM((CHUNK, d), jnp.float32),
            pltpu.VMEM((CHUNK, d), jnp.float32),
            pltpu.VMEM_SHARED((N_NODES, d), jnp.float32),
            pltpu.SemaphoreType.DMA,
            pltpu.SemaphoreType.DMA,
        ],
    )
    return kern(feat, srcp, dstp).reshape(NC, N_NODES, d)


ROWS_TC = 2000


def _scale_in_body(x_ref, w1_ref, degc_ref, xws_ref, dinv_ref):
    deg = jnp.maximum(degc_ref[0] + degc_ref[1], 1.0)
    dinv = lax.rsqrt(deg)
    dinv_ref[...] = dinv
    xw = jnp.dot(x_ref[...], w1_ref[...],
                 preferred_element_type=jnp.float32)
    xws_ref[...] = xw * dinv[:, 0:1]


def _mid_body(p_ref, dinv_ref, out_ref):
    dv = dinv_ref[:, 0:1]
    h = jnp.maximum((p_ref[0] + p_ref[1]) * dv, 0.0)
    out_ref[...] = h * dv


def _final_body(q_ref, dinv_ref, w2_ref, b2_ref, out_ref):
    z = (q_ref[0] + q_ref[1]) * dinv_ref[:, 0:1]
    logits = jnp.dot(z, w2_ref[...],
                     preferred_element_type=jnp.float32) + b2_ref[0:1, :]
    mx = jnp.max(logits, axis=1, keepdims=True)
    lse = jnp.log(jnp.sum(jnp.exp(logits - mx), axis=1, keepdims=True)) + mx
    out_ref[...] = logits - lse


def kernel(x, edge_index, W1, W2, b2):
    ei = edge_index.astype(jnp.int32)
    dst3 = ei[1].reshape(NW, DNCHUNK, DCHUNK)
    srcp = ei[0].reshape(NW, PAGES, PCHUNK, CHUNK)
    dstp = ei[1].reshape(NW, PAGES, PCHUNK, CHUNK)
    b2r = b2.reshape(1, NCLASS)

    degc = _deg(dst3)[:, :, :16]

    grid = (N_NODES // ROWS_TC,)
    xws, dinv16 = pl.pallas_call(
        _scale_in_body,
        grid=grid,
        in_specs=[
            pl.BlockSpec((ROWS_TC, NFEAT), lambda i: (i, 0)),
            pl.BlockSpec((NFEAT, NHID), lambda i: (0, 0)),
            pl.BlockSpec((NC, ROWS_TC, 16), lambda i: (0, i, 0)),
        ],
        out_specs=[
            pl.BlockSpec((ROWS_TC, NHID), lambda i: (i, 0)),
            pl.BlockSpec((ROWS_TC, 16), lambda i: (i, 0)),
        ],
        out_shape=[
            jax.ShapeDtypeStruct((N_NODES, NHID), jnp.float32),
            jax.ShapeDtypeStruct((N_NODES, 16), jnp.float32),
        ],
    )(x, W1, degc)

    p = _spmm(xws, srcp, dstp, NHID)

    hs = pl.pallas_call(
        _mid_body,
        grid=grid,
        in_specs=[
            pl.BlockSpec((NC, ROWS_TC, NHID), lambda i: (0, i, 0)),
            pl.BlockSpec((ROWS_TC, 16), lambda i: (i, 0)),
        ],
        out_specs=pl.BlockSpec((ROWS_TC, NHID), lambda i: (i, 0)),
        out_shape=jax.ShapeDtypeStruct((N_NODES, NHID), jnp.float32),
    )(p, dinv16)

    q = _spmm(hs, srcp, dstp, NHID)

    out = pl.pallas_call(
        _final_body,
        grid=grid,
        in_specs=[
            pl.BlockSpec((NC, ROWS_TC, NHID), lambda i: (0, i, 0)),
            pl.BlockSpec((ROWS_TC, 16), lambda i: (i, 0)),
            pl.BlockSpec((NHID, NCLASS), lambda i: (0, 0)),
            pl.BlockSpec((1, NCLASS), lambda i: (0, 0)),
        ],
        out_specs=pl.BlockSpec((ROWS_TC, NCLASS), lambda i: (i, 0)),
        out_shape=jax.ShapeDtypeStruct((N_NODES, NCLASS), jnp.float32),
    )(q, dinv16, W2, b2r)
    return out


# speedup vs baseline: 2.4739x; 1.0884x over previous
import functools

import jax
import jax.numpy as jnp
from jax import lax
from jax.experimental import pallas as pl
from jax.experimental.pallas import tpu as pltpu
from jax.experimental.pallas import tpu_sc as plsc

N_NODES = 10000
N_EDGES = 320000
NFEAT = 128
NHID = 128
NCLASS = 40

NC = 2
NS = 16
NW = NC * NS
EDGES_PER_TILE = N_EDGES // NW
DCHUNK = 80
DNCHUNK = EDGES_PER_TILE // DCHUNK
ROWS_PER_TILE = N_NODES // NS
ZR = 25

CHUNK = 80
NCHUNK = EDGES_PER_TILE // CHUNK
PAGES = 5
PCHUNK = NCHUNK // PAGES

_mesh = functools.partial(
    plsc.VectorSubcoreMesh, core_axis_name="c", subcore_axis_name="s")


def _zero_fill(zb, d):
    z16 = jnp.zeros((16,), jnp.float32)
    for r in range(ZR):
        for q in range(d // 16):
            zb[r, pl.ds(16 * q, 16)] = z16


def _zero_acc(zb, acc, s):
    for j in range(ROWS_PER_TILE // ZR):
        pltpu.sync_copy(zb, acc.at[pl.ds(s * ROWS_PER_TILE + j * ZR, ZR)])


def _copy_out(acc, out, c, s):
    pltpu.sync_copy(acc.at[pl.ds(s * ROWS_PER_TILE, ROWS_PER_TILE)],
                    out.at[c, s])


def _deg_body(dst_hbm, out_hbm, dst_v, ones_v, zb, acc, sem):
    c = lax.axis_index("c")
    s = lax.axis_index("s")
    wid = s * NC + c
    o16 = jnp.ones((16,), jnp.float32)
    for r in range(DCHUNK):
        for q in range(NHID // 16):
            ones_v[r, pl.ds(16 * q, 16)] = o16
    _zero_fill(zb, NHID)
    _zero_acc(zb, acc, s)
    pltpu.sync_copy(dst_hbm.at[wid], dst_v)
    plsc.subcore_barrier()

    def body(i, carry):
        pltpu.async_copy(ones_v, acc.at[dst_v.at[i]], sem, add=True)
        return carry

    lax.fori_loop(0, DNCHUNK, body, 0)

    def drain(i, carry):
        pltpu.make_async_copy(ones_v, acc.at[dst_v.at[i]], sem).wait()
        return carry

    lax.fori_loop(0, DNCHUNK, drain, 0)
    plsc.subcore_barrier()
    _copy_out(acc, out_hbm, c, s)


def _deg(dst3):
    kern = pl.kernel(
        _deg_body,
        out_type=jax.ShapeDtypeStruct((NC, NS, ROWS_PER_TILE, NHID),
                                      jnp.float32),
        mesh=_mesh(),
        scratch_types=[
            pltpu.VMEM((DNCHUNK, DCHUNK), jnp.int32),
            pltpu.VMEM((DCHUNK, NHID), jnp.float32),
            pltpu.VMEM((ZR, NHID), jnp.float32),
            pltpu.VMEM_SHARED((N_NODES, NHID), jnp.float32),
            pltpu.SemaphoreType.DMA,
        ],
    )
    return kern(dst3).reshape(NC, N_NODES, NHID)


def _spmm_body(d, feat_hbm, src_hbm, dst_hbm, out_hbm,
               src_pg, dst_pg, rows0, rows1, rows2, acc,
               sr0, sr1, sr2, sw0, sw1, sw2):
    c = lax.axis_index("c")
    s = lax.axis_index("s")
    wid = s * NC + c
    rows = [rows0, rows1, rows2]
    srs = [sr0, sr1, sr2]
    sws = [sw0, sw1, sw2]
    z16 = jnp.zeros((16,), jnp.float32)
    for r in range(ZR):
        for q in range(d // 16):
            rows0[r, pl.ds(16 * q, 16)] = z16
    for j in range(ROWS_PER_TILE // ZR):
        pltpu.sync_copy(rows0.at[pl.ds(0, ZR)],
                        acc.at[pl.ds(s * ROWS_PER_TILE + j * ZR, ZR)])
    plsc.subcore_barrier()

    def gath(i, b):
        pltpu.make_async_copy(
            feat_hbm.at[src_pg.at[i]], rows[b], srs[b]).start()

    def gath_wait(i, b):
        pltpu.make_async_copy(
            feat_hbm.at[src_pg.at[i]], rows[b], srs[b]).wait()

    def scat(i, b):
        pltpu.async_copy(rows[b], acc.at[dst_pg.at[i]], sws[b], add=True)

    def scat_wait(i, b):
        pltpu.make_async_copy(rows[b], acc.at[dst_pg.at[i]], sws[b]).wait()

    def step(i, unsafe_py_j):
        b = unsafe_py_j % 3
        gath_wait(i, b)
        scat(i, b)
        if unsafe_py_j >= 1:
            scat_wait(i - 1, (unsafe_py_j + 2) % 3)
        if unsafe_py_j <= PCHUNK - 3:
            gath(i + 2, (unsafe_py_j + 2) % 3)

    for p in range(PAGES):
        pltpu.sync_copy(src_hbm.at[wid, p], src_pg)
        pltpu.sync_copy(dst_hbm.at[wid, p], dst_pg)
        gath(0, 0)
        gath(1, 1)
        for i in range(PCHUNK):
            step(i, i)
        scat_wait(PCHUNK - 1, (PCHUNK - 1) % 3)

    plsc.subcore_barrier()
    _copy_out(acc, out_hbm, c, s)


def _spmm(feat, srcp, dstp, d):
    kern = pl.kernel(
        functools.partial(_spmm_body, d),
        out_type=jax.ShapeDtypeStruct((NC, NS, ROWS_PER_TILE, d),
                                      jnp.float32),
        mesh=_mesh(),
        scratch_types=[
            pltpu.VMEM((PCHUNK, CHUNK), jnp.int32),
            pltpu.VMEM((PCHUNK, CHUNK), jnp.int32),
            pltpu.VMEM((CHUNK, d), jnp.float32),
            pltpu.VMEM((CHUNK, d), jnp.float32),
            pltpu.VMEM((CHUNK, d), jnp.float32),
            pltpu.VMEM_SHARED((N_NODES, d), jnp.float32),
            pltpu.SemaphoreType.DMA,
            pltpu.SemaphoreType.DMA,
            pltpu.SemaphoreType.DMA,
            pltpu.SemaphoreType.DMA,
            pltpu.SemaphoreType.DMA,
            pltpu.SemaphoreType.DMA,
        ],
    )
    return kern(feat, srcp, dstp).reshape(NC, N_NODES, d)


ROWS_TC = 2000


def _scale_in_body(x_ref, w1_ref, degc_ref, xws_ref, dinv_ref):
    deg = jnp.maximum(degc_ref[0] + degc_ref[1], 1.0)
    dinv = lax.rsqrt(deg)
    dinv_ref[...] = dinv
    xw = jnp.dot(x_ref[...], w1_ref[...],
                 preferred_element_type=jnp.float32)
    xws_ref[...] = xw * dinv[:, 0:1]


def _mid_body(p_ref, dinv_ref, out_ref):
    dv = dinv_ref[:, 0:1]
    h = jnp.maximum((p_ref[0] + p_ref[1]) * dv, 0.0)
    out_ref[...] = h * dv


def _final_body(q_ref, dinv_ref, w2_ref, b2_ref, out_ref):
    z = (q_ref[0] + q_ref[1]) * dinv_ref[:, 0:1]
    logits = jnp.dot(z, w2_ref[...],
                     preferred_element_type=jnp.float32) + b2_ref[0:1, :]
    mx = jnp.max(logits, axis=1, keepdims=True)
    lse = jnp.log(jnp.sum(jnp.exp(logits - mx), axis=1, keepdims=True)) + mx
    out_ref[...] = logits - lse


def kernel(x, edge_index, W1, W2, b2):
    ei = edge_index.astype(jnp.int32)
    dst3 = ei[1].reshape(NW, DNCHUNK, DCHUNK)
    srcp = ei[0].reshape(NW, PAGES, PCHUNK, CHUNK)
    dstp = ei[1].reshape(NW, PAGES, PCHUNK, CHUNK)
    b2r = b2.reshape(1, NCLASS)

    degc = _deg(dst3)[:, :, :16]

    grid = (N_NODES // ROWS_TC,)
    xws, dinv16 = pl.pallas_call(
        _scale_in_body,
        grid=grid,
        in_specs=[
            pl.BlockSpec((ROWS_TC, NFEAT), lambda i: (i, 0)),
            pl.BlockSpec((NFEAT, NHID), lambda i: (0, 0)),
            pl.BlockSpec((NC, ROWS_TC, 16), lambda i: (0, i, 0)),
        ],
        out_specs=[
            pl.BlockSpec((ROWS_TC, NHID), lambda i: (i, 0)),
            pl.BlockSpec((ROWS_TC, 16), lambda i: (i, 0)),
        ],
        out_shape=[
            jax.ShapeDtypeStruct((N_NODES, NHID), jnp.float32),
            jax.ShapeDtypeStruct((N_NODES, 16), jnp.float32),
        ],
    )(x, W1, degc)

    p = _spmm(xws, srcp, dstp, NHID)

    hs = pl.pallas_call(
        _mid_body,
        grid=grid,
        in_specs=[
            pl.BlockSpec((NC, ROWS_TC, NHID), lambda i: (0, i, 0)),
            pl.BlockSpec((ROWS_TC, 16), lambda i: (i, 0)),
        ],
        out_specs=pl.BlockSpec((ROWS_TC, NHID), lambda i: (i, 0)),
        out_shape=jax.ShapeDtypeStruct((N_NODES, NHID), jnp.float32),
    )(p, dinv16)

    q = _spmm(hs, srcp, dstp, NHID)

    out = pl.pallas_call(
        _final_body,
        grid=grid,
        in_specs=[
            pl.BlockSpec((NC, ROWS_TC, NHID), lambda i: (0, i, 0)),
            pl.BlockSpec((ROWS_TC, 16), lambda i: (i, 0)),
            pl.BlockSpec((NHID, NCLASS), lambda i: (0, 0)),
            pl.BlockSpec((1, NCLASS), lambda i: (0, 0)),
        ],
        out_specs=pl.BlockSpec((ROWS_TC, NCLASS), lambda i: (i, 0)),
        out_shape=jax.ShapeDtypeStruct((N_NODES, NCLASS), jnp.float32),
    )(q, dinv16, W2, b2r)
    return out
